# baseline (speedup 1.0000x reference)
import functools

import jax
import jax.numpy as jnp
from jax import lax
from jax.experimental import pallas as pl
from jax.experimental.pallas import tpu as pltpu
from jax.experimental.pallas import tpu_sc as plsc

_T, _B, _D, _L = 20, 32, 128, 3
_TB = _T * _B
_G4 = 4 * _D


@functools.lru_cache(maxsize=None)
def _make_sc_gather(vocab, d, n_idx):
    info = plsc.get_sparse_core_info()
    nc, ns = info.num_cores, info.num_subcores
    nw = nc * ns
    b_per_w = 32
    n_active = n_idx // b_per_w
    assert n_idx % b_per_w == 0 and n_active <= nw
    mesh = plsc.VectorSubcoreMesh(core_axis_name="c", subcore_axis_name="s")

    @functools.partial(
        pl.kernel,
        mesh=mesh,
        out_type=jax.ShapeDtypeStruct((n_idx, d), jnp.float32),
        scratch_types=[
            pltpu.VMEM((b_per_w,), jnp.int32),
            pltpu.VMEM((b_per_w, d), jnp.float32),
            pltpu.SemaphoreType.DMA,
        ],
    )
    def gather_k(table_hbm, idx_hbm, out_hbm, idx_v, rows_v, sem):
        wid = lax.axis_index("s") * nc + lax.axis_index("c")

        @pl.when(wid < n_active)
        def _():
            base = wid * b_per_w
            pltpu.sync_copy(idx_hbm.at[pl.ds(base, b_per_w)], idx_v)
            pltpu.async_copy(table_hbm.at[idx_v], rows_v, sem).wait()
            pltpu.sync_copy(rows_v, out_hbm.at[pl.ds(base, b_per_w)])

    return gather_k


_LD = _L * _D


def _lstm_body(x_ref, h0_ref, c0_ref, wcat_ref, bcat_ref,
               ys_ref, ht_ref, ct_ref):
    h = [h0_ref[l] for l in range(_L)]
    c = [c0_ref[l] for l in range(_L)]
    wcat = wcat_ref[:]
    bcat = bcat_ref[:]
    dn = (((1,), (0,)), ((), ()))
    for s in range(_T + _L - 1):
        ts = min(s, _T - 1)
        x_t = x_ref[ts * _B:(ts + 1) * _B, :]
        u = jnp.concatenate([h[0], h[1], h[2], x_t], axis=1)
        gates = lax.dot_general(
            u, wcat, dn, preferred_element_type=jnp.float32) + bcat
        sig_if = jax.nn.sigmoid(gates[:, :2 * _LD])
        g_all = jnp.tanh(gates[:, 2 * _LD:3 * _LD])
        o_all = jax.nn.sigmoid(gates[:, 3 * _LD:])
        for l in range(_L):
            t = s - l
            if 0 <= t < _T:
                i_l = sig_if[:, _D * l:_D * (l + 1)]
                f_l = sig_if[:, _LD + _D * l:_LD + _D * (l + 1)]
                g_l = g_all[:, _D * l:_D * (l + 1)]
                o_l = o_all[:, _D * l:_D * (l + 1)]
                c[l] = f_l * c[l] + i_l * g_l
                h[l] = o_l * jnp.tanh(c[l])
                if l == _L - 1:
                    ys_ref[t * _B:(t + 1) * _B, :] = h[l]
    for l in range(_L):
        ht_ref[l] = h[l]
        ct_ref[l] = c[l]


def _pack_lstm_weights(wihs, whhs, bihs, bhhs):
    wcat = jnp.zeros((_LD + _D, 4 * _LD), dtype=jnp.float32)
    bcat = jnp.zeros((1, 4 * _LD), dtype=jnp.float32)
    for k in range(4):
        for l in range(_L):
            col = _LD * k + _D * l
            wcat = wcat.at[_D * l:_D * (l + 1), col:col + _D].set(
                whhs[l][_D * k:_D * (k + 1), :].T)
            row = _D * (l - 1) if l > 0 else _LD
            wcat = wcat.at[row:row + _D, col:col + _D].set(
                wihs[l][_D * k:_D * (k + 1), :].T)
            bcat = bcat.at[0, col:col + _D].set(
                bihs[l][_D * k:_D * (k + 1)] + bhhs[l][_D * k:_D * (k + 1)])
    return wcat, bcat


def _run_lstm(xs, h0, c0, wcat, bcat):
    out_shapes = (
        jax.ShapeDtypeStruct((_TB, _D), jnp.float32),
        jax.ShapeDtypeStruct((_L, _B, _D), jnp.float32),
        jax.ShapeDtypeStruct((_L, _B, _D), jnp.float32),
    )
    return pl.pallas_call(
        _lstm_body,
        out_shape=out_shapes,
    )(xs, h0, c0, wcat, bcat)


_V_TILE = 8192


def _proj_body(a_ref, w_ref, b_ref, o_ref):
    o_ref[:] = lax.dot_general(
        a_ref[:], w_ref[:], (((1,), (1,)), ((), ())),
        preferred_element_type=jnp.float32) + b_ref[:]


def _run_proj(ys, fc_w, fc_b2d, vocab):
    grid = (pl.cdiv(vocab, _V_TILE),)
    return pl.pallas_call(
        _proj_body,
        grid=grid,
        in_specs=[
            pl.BlockSpec((_TB, _D), lambda i: (0, 0)),
            pl.BlockSpec((_V_TILE, _D), lambda i: (i, 0)),
            pl.BlockSpec((1, _V_TILE), lambda i: (0, i)),
        ],
        out_specs=pl.BlockSpec((_TB, _V_TILE), lambda i: (0, i)),
        out_shape=jax.ShapeDtypeStruct((_TB, vocab), jnp.float32),
        compiler_params=pltpu.CompilerParams(
            dimension_semantics=("arbitrary",)),
    )(ys, fc_w, fc_b2d)


def kernel(x, h0, c0, emb,
           W_ih0, W_hh0, b_ih0, b_hh0,
           W_ih1, W_hh1, b_ih1, b_hh1,
           W_ih2, W_hh2, b_ih2, b_hh2,
           fc_W, fc_b):
    vocab = emb.shape[0]
    idx = x.reshape(_TB)
    gathered = _make_sc_gather(vocab, _D, _TB)(emb, idx)

    wcat, bcat = _pack_lstm_weights(
        (W_ih0, W_ih1, W_ih2), (W_hh0, W_hh1, W_hh2),
        (b_ih0, b_ih1, b_ih2), (b_hh0, b_hh1, b_hh2))
    ys, ht, ct = _run_lstm(gathered, h0, c0, wcat, bcat)

    logits = _run_proj(ys, fc_W, fc_b.reshape(1, vocab), vocab)
    return logits.reshape(_T, _B, vocab), (ht, ct)


# speedup vs baseline: 1.3598x; 1.3598x over previous
import functools

import jax
import jax.numpy as jnp
from jax import lax
from jax.experimental import pallas as pl
from jax.experimental.pallas import tpu as pltpu
from jax.experimental.pallas import tpu_sc as plsc

_T, _B, _D, _L = 20, 32, 128, 3
_TB = _T * _B
_G4 = 4 * _D
_VOCAB = 100000
_VT = 1024
_NFULL = _VOCAB // _VT
_REM = _VOCAB - _NFULL * _VT
_REM_PAD = ((_REM + 127) // 128) * 128
_VPAD = _NFULL * _VT + _REM_PAD
_NT = _NFULL + 1
_NW = 10
_WROWS = _VOCAB // _NW


@functools.lru_cache(maxsize=None)
def _make_sc_gather(vocab, d, n_idx):
    info = plsc.get_sparse_core_info()
    nc, ns = info.num_cores, info.num_subcores
    nw = nc * ns
    b_per_w = 32
    n_active = n_idx // b_per_w
    assert n_idx % b_per_w == 0 and n_active <= nw
    mesh = plsc.VectorSubcoreMesh(core_axis_name="c", subcore_axis_name="s")

    @functools.partial(
        pl.kernel,
        mesh=mesh,
        out_type=jax.ShapeDtypeStruct((n_idx, d), jnp.float32),
        scratch_types=[
            pltpu.VMEM((b_per_w,), jnp.int32),
            pltpu.VMEM((b_per_w, d), jnp.float32),
            pltpu.SemaphoreType.DMA,
        ],
    )
    def gather_k(table_hbm, idx_hbm, out_hbm, idx_v, rows_v, sem):
        wid = lax.axis_index("s") * nc + lax.axis_index("c")

        @pl.when(wid < n_active)
        def _():
            base = wid * b_per_w
            pltpu.sync_copy(idx_hbm.at[pl.ds(base, b_per_w)], idx_v)
            pltpu.async_copy(table_hbm.at[idx_v], rows_v, sem).wait()
            pltpu.sync_copy(rows_v, out_hbm.at[pl.ds(base, b_per_w)])

    return gather_k


def _w_copy(i, fcw_hbm, wf_ref, semw):
    return pltpu.make_async_copy(
        fcw_hbm.at[pl.ds(i * _WROWS, _WROWS)],
        wf_ref.at[pl.ds(i * _WROWS, _WROWS)],
        semw.at[i])


def _fused_body(x_ref, h0_ref, c0_ref,
                wih0, whh0, bih0, bhh0,
                wih1, whh1, bih1, bhh1,
                wih2, whh2, bih2, bhh2,
                fcw_hbm, fcb_ref,
                logits_ref, ht_ref, ct_ref,
                ys_ref, gx_ref, wf_ref, semw):
    step = pl.program_id(0)

    @pl.when(step == 0)
    def _prologue():
        for i in range(_NW):
            _w_copy(i, fcw_hbm, wf_ref, semw).start()

        params = ((wih0, whh0, bih0, bhh0),
                  (wih1, whh1, bih1, bhh1),
                  (wih2, whh2, bih2, bhh2))
        dn = (((1,), (1,)), ((), ()))
        for l in range(_L):
            wih, whh, bih, bhh = params[l]
            src = x_ref if l == 0 else ys_ref
            gx_ref[:] = (
                lax.dot_general(src[:], wih[:], dn,
                                preferred_element_type=jnp.float32)
                + bih[:] + bhh[:]
            )
            h = h0_ref[l]
            c = c0_ref[l]
            whh_v = whh[:]
            for t in range(_T):
                gates = gx_ref[t * _B:(t + 1) * _B, :] + lax.dot_general(
                    h, whh_v, dn, preferred_element_type=jnp.float32)
                i_g = jax.nn.sigmoid(gates[:, :_D])
                f_g = jax.nn.sigmoid(gates[:, _D:2 * _D])
                g_g = jnp.tanh(gates[:, 2 * _D:3 * _D])
                o_g = jax.nn.sigmoid(gates[:, 3 * _D:])
                c = f_g * c + i_g * g_g
                h = o_g * jnp.tanh(c)
                ys_ref[t * _B:(t + 1) * _B, :] = h
            ht_ref[l] = h
            ct_ref[l] = c

        for i in range(_NW):
            _w_copy(i, fcw_hbm, wf_ref, semw).wait()

    dnp = (((1,), (1,)), ((), ()))

    @pl.when(step < _NFULL)
    def _full_tile():
        base = pl.multiple_of(step * _VT, _VT)
        wv = wf_ref[pl.ds(base, _VT), :]
        logits_ref[:] = lax.dot_general(
            ys_ref[:], wv, dnp,
            preferred_element_type=jnp.float32) + fcb_ref[:]

    @pl.when(step == _NFULL)
    def _ragged_tile():
        wv = wf_ref[pl.ds(_NFULL * _VT, _REM_PAD), :]
        logits_ref[:, :_REM_PAD] = lax.dot_general(
            ys_ref[:], wv, dnp,
            preferred_element_type=jnp.float32) + fcb_ref[:, :_REM_PAD]


def _run_fused(xs, h0, c0, ws, fc_w, fcb_pad):
    vfull = pl.BlockSpec(memory_space=pltpu.MemorySpace.VMEM)
    hspec = pl.BlockSpec(memory_space=pltpu.MemorySpace.HBM)
    out_shapes = (
        jax.ShapeDtypeStruct((_TB, _VOCAB), jnp.float32),
        jax.ShapeDtypeStruct((_L, _B, _D), jnp.float32),
        jax.ShapeDtypeStruct((_L, _B, _D), jnp.float32),
    )
    return pl.pallas_call(
        _fused_body,
        grid=(_NT,),
        in_specs=[vfull] * 15 + [
            hspec,
            pl.BlockSpec((1, _VT), lambda i: (0, i)),
        ],
        out_specs=(
            pl.BlockSpec((_TB, _VT), lambda i: (0, i)),
            pl.BlockSpec((_L, _B, _D), lambda i: (0, 0, 0)),
            pl.BlockSpec((_L, _B, _D), lambda i: (0, 0, 0)),
        ),
        out_shape=out_shapes,
        scratch_shapes=[
            pltpu.VMEM((_TB, _D), jnp.float32),
            pltpu.VMEM((_TB, _G4), jnp.float32),
            pltpu.VMEM((_VPAD, _D), jnp.float32),
            pltpu.SemaphoreType.DMA((_NW,)),
        ],
        compiler_params=pltpu.CompilerParams(
            dimension_semantics=("arbitrary",)),
    )(xs, h0, c0, *ws, fc_w, fcb_pad)


def kernel(x, h0, c0, emb,
           W_ih0, W_hh0, b_ih0, b_hh0,
           W_ih1, W_hh1, b_ih1, b_hh1,
           W_ih2, W_hh2, b_ih2, b_hh2,
           fc_W, fc_b):
    vocab = emb.shape[0]
    idx = x.reshape(_TB)
    gathered = _make_sc_gather(vocab, _D, _TB)(emb, idx)

    ws = (W_ih0, W_hh0, b_ih0.reshape(1, _G4), b_hh0.reshape(1, _G4),
          W_ih1, W_hh1, b_ih1.reshape(1, _G4), b_hh1.reshape(1, _G4),
          W_ih2, W_hh2, b_ih2.reshape(1, _G4), b_hh2.reshape(1, _G4))
    logits, ht, ct = _run_fused(gathered, h0, c0, ws, fc_W,
                                fc_b.reshape(1, vocab))
    return logits.reshape(_T, _B, vocab), (ht, ct)


# speedup vs baseline: 1.6114x; 1.1851x over previous
import functools

import jax
import jax.numpy as jnp
from jax import lax
from jax.experimental import pallas as pl
from jax.experimental.pallas import tpu as pltpu
from jax.experimental.pallas import tpu_sc as plsc

_T, _B, _D, _L = 20, 32, 128, 3
_TB = _T * _B
_G4 = 4 * _D
_VOCAB = 100000
_VT = 2048
_NT = (_VOCAB + _VT - 1) // _VT
_KRES = 42
_RROWS = _KRES * _VT
_NW = 6
_WROWS = _RROWS // _NW


@functools.lru_cache(maxsize=None)
def _make_sc_gather(vocab, d, n_idx):
    info = plsc.get_sparse_core_info()
    nc, ns = info.num_cores, info.num_subcores
    nw = nc * ns
    b_per_w = 32
    n_active = n_idx // b_per_w
    assert n_idx % b_per_w == 0 and n_active <= nw
    mesh = plsc.VectorSubcoreMesh(core_axis_name="c", subcore_axis_name="s")

    @functools.partial(
        pl.kernel,
        mesh=mesh,
        out_type=jax.ShapeDtypeStruct((n_idx, d), jnp.float32),
        scratch_types=[
            pltpu.VMEM((b_per_w,), jnp.int32),
            pltpu.VMEM((b_per_w, d), jnp.float32),
            pltpu.SemaphoreType.DMA,
        ],
    )
    def gather_k(table_hbm, idx_hbm, out_hbm, idx_v, rows_v, sem):
        wid = lax.axis_index("s") * nc + lax.axis_index("c")

        @pl.when(wid < n_active)
        def _():
            base = wid * b_per_w
            pltpu.sync_copy(idx_hbm.at[pl.ds(base, b_per_w)], idx_v)
            pltpu.async_copy(table_hbm.at[idx_v], rows_v, sem).wait()
            pltpu.sync_copy(rows_v, out_hbm.at[pl.ds(base, b_per_w)])

    return gather_k


def _w_copy(i, fcw_hbm, wf_ref, semw):
    return pltpu.make_async_copy(
        fcw_hbm.at[pl.ds(i * _WROWS, _WROWS)],
        wf_ref.at[pl.ds(i * _WROWS, _WROWS)],
        semw.at[i])


def _fused_body(x_ref, h0_ref, c0_ref,
                wih0, whh0, bih0, bhh0,
                wih1, whh1, bih1, bhh1,
                wih2, whh2, bih2, bhh2,
                fcw_hbm, wstream_ref, fcb_ref,
                logits_ref, ht_ref, ct_ref,
                ys_ref, gx_ref, wf_ref, semw):
    step = pl.program_id(0)

    @pl.when(step == 0)
    def _prologue():
        for i in range(_NW):
            _w_copy(i, fcw_hbm, wf_ref, semw).start()

        params = ((wih0, whh0, bih0, bhh0),
                  (wih1, whh1, bih1, bhh1),
                  (wih2, whh2, bih2, bhh2))
        dn = (((1,), (1,)), ((), ()))
        for l in range(_L):
            wih, whh, bih, bhh = params[l]
            src = x_ref if l == 0 else ys_ref
            gx_ref[:] = (
                lax.dot_general(src[:], wih[:], dn,
                                preferred_element_type=jnp.float32)
                + bih[:] + bhh[:]
            )
            h = h0_ref[l]
            c = c0_ref[l]
            whh_v = whh[:]
            for t in range(_T):
                gates = gx_ref[t * _B:(t + 1) * _B, :] + lax.dot_general(
                    h, whh_v, dn, preferred_element_type=jnp.float32)
                i_g = jax.nn.sigmoid(gates[:, :_D])
                f_g = jax.nn.sigmoid(gates[:, _D:2 * _D])
                g_g = jnp.tanh(gates[:, 2 * _D:3 * _D])
                o_g = jax.nn.sigmoid(gates[:, 3 * _D:])
                c = f_g * c + i_g * g_g
                h = o_g * jnp.tanh(c)
                ys_ref[t * _B:(t + 1) * _B, :] = h
            ht_ref[l] = h
            ct_ref[l] = c

        for i in range(_NW):
            _w_copy(i, fcw_hbm, wf_ref, semw).wait()

    dnp = (((1,), (1,)), ((), ()))

    @pl.when(step < _KRES)
    def _resident_tile():
        base = pl.multiple_of(step * _VT, _VT)
        wv = wf_ref[pl.ds(base, _VT), :]
        logits_ref[:] = lax.dot_general(
            ys_ref[:], wv, dnp,
            preferred_element_type=jnp.float32) + fcb_ref[:]

    @pl.when(step >= _KRES)
    def _streamed_tile():
        logits_ref[:] = lax.dot_general(
            ys_ref[:], wstream_ref[:], dnp,
            preferred_element_type=jnp.float32) + fcb_ref[:]


def _run_fused(xs, h0, c0, ws, fc_w, fcb_pad):
    vfull = pl.BlockSpec(memory_space=pltpu.MemorySpace.VMEM)
    hspec = pl.BlockSpec(memory_space=pltpu.MemorySpace.HBM)
    out_shapes = (
        jax.ShapeDtypeStruct((_TB, _VOCAB), jnp.float32),
        jax.ShapeDtypeStruct((_L, _B, _D), jnp.float32),
        jax.ShapeDtypeStruct((_L, _B, _D), jnp.float32),
    )
    return pl.pallas_call(
        _fused_body,
        grid=(_NT,),
        in_specs=[vfull] * 15 + [
            hspec,
            pl.BlockSpec((_VT, _D),
                         lambda i: (jnp.maximum(i, _KRES), 0)),
            pl.BlockSpec((1, _VT), lambda i: (0, i)),
        ],
        out_specs=(
            pl.BlockSpec((_TB, _VT), lambda i: (0, i)),
            pl.BlockSpec((_L, _B, _D), lambda i: (0, 0, 0)),
            pl.BlockSpec((_L, _B, _D), lambda i: (0, 0, 0)),
        ),
        out_shape=out_shapes,
        scratch_shapes=[
            pltpu.VMEM((_TB, _D), jnp.float32),
            pltpu.VMEM((_TB, _G4), jnp.float32),
            pltpu.VMEM((_RROWS, _D), jnp.float32),
            pltpu.SemaphoreType.DMA((_NW,)),
        ],
        compiler_params=pltpu.CompilerParams(
            dimension_semantics=("arbitrary",)),
    )(xs, h0, c0, *ws, fc_w, fc_w, fcb_pad)


def kernel(x, h0, c0, emb,
           W_ih0, W_hh0, b_ih0, b_hh0,
           W_ih1, W_hh1, b_ih1, b_hh1,
           W_ih2, W_hh2, b_ih2, b_hh2,
           fc_W, fc_b):
    vocab = emb.shape[0]
    idx = x.reshape(_TB)
    gathered = _make_sc_gather(vocab, _D, _TB)(emb, idx)

    ws = (W_ih0, W_hh0, b_ih0.reshape(1, _G4), b_hh0.reshape(1, _G4),
          W_ih1, W_hh1, b_ih1.reshape(1, _G4), b_hh1.reshape(1, _G4),
          W_ih2, W_hh2, b_ih2.reshape(1, _G4), b_hh2.reshape(1, _G4))
    logits, ht, ct = _run_fused(gathered, h0, c0, ws, fc_W,
                                fc_b.reshape(1, vocab))
    return logits.reshape(_T, _B, vocab), (ht, ct)
